# Initial kernel scaffold; baseline (speedup 1.0000x reference)
import jax
import jax.numpy as jnp
from jax.experimental import pallas as pl


def kernel(x, embedding_sum, cluster_usage):
    raise NotImplementedError("write your pallas kernel here")


# speedup vs baseline: 1.0180x; 1.0180x over previous
import functools

import jax
import jax.numpy as jnp
from jax.experimental import pallas as pl
from jax.experimental.pallas import tpu as pltpu

SEM_DIM = 256
AC_DIM = 36
SEM_K = 8192
AC_LEVELS = 21
EPS = 1e-05

TOK_BLK = 512
ARGMIN_WIN = 2048


def _vq_kernel(xs_ref, ac_ref, eb_ref, e2_ref, x2_ref,
               sem_out_ref, ac_out_ref):
    t = jnp.tanh(ac_ref[...])
    scaled = (t + 1.0) * (0.5 * (AC_LEVELS - 1))
    ac_out_ref[...] = jnp.round(scaled).astype(jnp.int32)

    xb = xs_ref[0]
    a2 = (2.0 * xb).astype(jnp.bfloat16).astype(jnp.float32)
    x2 = x2_ref[0, 0]

    accv = jnp.full((TOK_BLK,), jnp.inf, jnp.float32)
    acci = jnp.zeros((TOK_BLK,), jnp.int32)
    num_win = (SEM_K + ARGMIN_WIN - 1) // ARGMIN_WIN
    for w in range(num_win):
        lo = w * ARGMIN_WIN
        wlen = min(ARGMIN_WIN, SEM_K - lo)
        ebw = eb_ref[pl.ds(lo, wlen), :]
        g2 = jax.lax.dot_general(
            ebw, a2, (((1,), (0,)), ((), ())),
            preferred_element_type=jnp.float32,
        )
        e2w = e2_ref[0, pl.ds(lo, wlen)]
        d2 = (x2[None, :] + e2w[:, None]) - g2
        dist = jnp.sqrt(jnp.maximum(0.0, d2))
        m = jnp.min(dist, axis=0)
        ids = jax.lax.broadcasted_iota(jnp.int32, dist.shape, 0)
        i = jnp.min(jnp.where(dist == m[None, :], ids, SEM_K), axis=0) + lo
        upd = m < accv
        accv = jnp.where(upd, m, accv)
        acci = jnp.where(upd, i, acci)
        if w != num_win - 1:
            accv = accv.astype(jnp.bfloat16).astype(jnp.float32)

    sem_out_ref[...] = acci.reshape(1, 1, TOK_BLK)


@jax.jit
def kernel(x, embedding_sum, cluster_usage):
    B, D, T = x.shape
    xs = x[:, :SEM_DIM, :]
    ac = x[:, SEM_DIM:, :]

    emb = embedding_sum / jnp.clip(cluster_usage, EPS, None)[:, None]
    flat = jnp.transpose(xs, (0, 2, 1)).reshape(B * T, SEM_DIM)
    x2 = jnp.sum(flat * flat, axis=1)
    e2 = jnp.sum(emb * emb, axis=1)
    ebf = emb

    toks_per_row = T // TOK_BLK
    num_tok_blocks = (B * T) // TOK_BLK
    x2_3d = x2.reshape(num_tok_blocks, 1, TOK_BLK)
    e2_2d = e2.reshape(1, SEM_K)

    sem_codes, ac_codes = pl.pallas_call(
        _vq_kernel,
        grid=(num_tok_blocks,),
        in_specs=[
            pl.BlockSpec((1, SEM_DIM, TOK_BLK),
                         lambda t: (t // toks_per_row, 0, t % toks_per_row)),
            pl.BlockSpec((1, AC_DIM, TOK_BLK),
                         lambda t: (t // toks_per_row, 0, t % toks_per_row)),
            pl.BlockSpec((SEM_K, SEM_DIM), lambda t: (0, 0)),
            pl.BlockSpec((1, SEM_K), lambda t: (0, 0)),
            pl.BlockSpec((1, 1, TOK_BLK), lambda t: (t, 0, 0)),
        ],
        out_specs=[
            pl.BlockSpec((1, 1, TOK_BLK),
                         lambda t: (t // toks_per_row, 0, t % toks_per_row)),
            pl.BlockSpec((1, AC_DIM, TOK_BLK),
                         lambda t: (t // toks_per_row, 0, t % toks_per_row)),
        ],
        out_shape=[
            jax.ShapeDtypeStruct((B, 1, T), jnp.int32),
            jax.ShapeDtypeStruct((B, AC_DIM, T), jnp.int32),
        ],
    )(xs, ac, ebf, e2_2d, x2_3d)

    return jnp.concatenate([sem_codes, ac_codes], axis=1)


# speedup vs baseline: 1.0412x; 1.0227x over previous
import functools

import jax
import jax.numpy as jnp
from jax.experimental import pallas as pl
from jax.experimental.pallas import tpu as pltpu

SEM_DIM = 256
AC_DIM = 36
SEM_K = 8192
AC_LEVELS = 21
EPS = 1e-05

TOK_BLK = 1024
ARGMIN_WIN = 2048


def _vq_kernel(xs_ref, ac_ref, eb_ref, e2_ref, x2_ref,
               sem_out_ref, ac_out_ref):
    t = jnp.tanh(ac_ref[...])
    scaled = (t + 1.0) * (0.5 * (AC_LEVELS - 1))
    ac_out_ref[...] = jnp.round(scaled).astype(jnp.int32)

    xb = xs_ref[0]
    a2 = (2.0 * xb).astype(jnp.bfloat16).astype(jnp.float32)
    x2 = x2_ref[0, 0]

    accv = jnp.full((TOK_BLK,), jnp.inf, jnp.float32)
    acci = jnp.zeros((TOK_BLK,), jnp.int32)
    num_win = (SEM_K + ARGMIN_WIN - 1) // ARGMIN_WIN
    for w in range(num_win):
        lo = w * ARGMIN_WIN
        wlen = min(ARGMIN_WIN, SEM_K - lo)
        ebw = eb_ref[pl.ds(lo, wlen), :]
        g2 = jax.lax.dot_general(
            ebw, a2, (((1,), (0,)), ((), ())),
            preferred_element_type=jnp.float32,
        )
        e2w = e2_ref[0, pl.ds(lo, wlen)]
        d2 = (x2[None, :] + e2w[:, None]) - g2
        dist = jnp.sqrt(jnp.maximum(0.0, d2))
        m = jnp.min(dist, axis=0)
        ids = jax.lax.broadcasted_iota(jnp.int32, dist.shape, 0)
        i = jnp.min(jnp.where(dist == m[None, :], ids, SEM_K), axis=0) + lo
        upd = m < accv
        accv = jnp.where(upd, m, accv)
        acci = jnp.where(upd, i, acci)
        if w != num_win - 1:
            accv = accv.astype(jnp.bfloat16).astype(jnp.float32)

    sem_out_ref[...] = acci.reshape(1, 1, TOK_BLK)


@jax.jit
def kernel(x, embedding_sum, cluster_usage):
    B, D, T = x.shape
    xs = x[:, :SEM_DIM, :]
    ac = x[:, SEM_DIM:, :]

    emb = embedding_sum / jnp.clip(cluster_usage, EPS, None)[:, None]
    flat = jnp.transpose(xs, (0, 2, 1)).reshape(B * T, SEM_DIM)
    x2 = jnp.sum(flat * flat, axis=1)
    e2 = jnp.sum(emb * emb, axis=1)
    ebf = emb

    toks_per_row = T // TOK_BLK
    num_tok_blocks = (B * T) // TOK_BLK
    x2_3d = x2.reshape(num_tok_blocks, 1, TOK_BLK)
    e2_2d = e2.reshape(1, SEM_K)

    sem_codes, ac_codes = pl.pallas_call(
        _vq_kernel,
        grid=(num_tok_blocks,),
        in_specs=[
            pl.BlockSpec((1, SEM_DIM, TOK_BLK),
                         lambda t: (t // toks_per_row, 0, t % toks_per_row)),
            pl.BlockSpec((1, AC_DIM, TOK_BLK),
                         lambda t: (t // toks_per_row, 0, t % toks_per_row)),
            pl.BlockSpec((SEM_K, SEM_DIM), lambda t: (0, 0)),
            pl.BlockSpec((1, SEM_K), lambda t: (0, 0)),
            pl.BlockSpec((1, 1, TOK_BLK), lambda t: (t, 0, 0)),
        ],
        out_specs=[
            pl.BlockSpec((1, 1, TOK_BLK),
                         lambda t: (t // toks_per_row, 0, t % toks_per_row)),
            pl.BlockSpec((1, AC_DIM, TOK_BLK),
                         lambda t: (t // toks_per_row, 0, t % toks_per_row)),
        ],
        out_shape=[
            jax.ShapeDtypeStruct((B, 1, T), jnp.int32),
            jax.ShapeDtypeStruct((B, AC_DIM, T), jnp.int32),
        ],
    )(xs, ac, ebf, e2_2d, x2_3d)

    return jnp.concatenate([sem_codes, ac_codes], axis=1)


# speedup vs baseline: 1.0470x; 1.0056x over previous
import functools

import jax
import jax.numpy as jnp
from jax.experimental import pallas as pl
from jax.experimental.pallas import tpu as pltpu

SEM_DIM = 256
AC_DIM = 36
SEM_K = 8192
AC_LEVELS = 21
EPS = 1e-05

TOK_BLK = 1024
ARGMIN_WIN = 2048


def _vq_kernel(x_ref, eb_ref, e2_ref, x2_ref,
               sem_out_ref, ac_out_ref):
    t = jnp.tanh(x_ref[0, SEM_DIM:])
    scaled = (t + 1.0) * (0.5 * (AC_LEVELS - 1))
    ac_out_ref[...] = jnp.round(scaled).astype(jnp.int32).reshape(
        1, AC_DIM, TOK_BLK)

    xb = x_ref[0, :SEM_DIM]
    a2 = (2.0 * xb).astype(jnp.bfloat16).astype(jnp.float32)
    x2 = x2_ref[0, 0]

    accv = jnp.full((TOK_BLK,), jnp.inf, jnp.float32)
    acci = jnp.zeros((TOK_BLK,), jnp.int32)
    num_win = (SEM_K + ARGMIN_WIN - 1) // ARGMIN_WIN
    for w in range(num_win):
        lo = w * ARGMIN_WIN
        wlen = min(ARGMIN_WIN, SEM_K - lo)
        ebw = eb_ref[pl.ds(lo, wlen), :]
        g2 = jax.lax.dot_general(
            ebw, a2, (((1,), (0,)), ((), ())),
            preferred_element_type=jnp.float32,
        )
        e2w = e2_ref[0, pl.ds(lo, wlen)]
        d2 = (x2[None, :] + e2w[:, None]) - g2
        dist = jnp.sqrt(jnp.maximum(0.0, d2))
        m = jnp.min(dist, axis=0)
        ids = jax.lax.broadcasted_iota(jnp.int32, dist.shape, 0)
        i = jnp.min(jnp.where(dist == m[None, :], ids, SEM_K), axis=0) + lo
        upd = m < accv
        accv = jnp.where(upd, m, accv)
        acci = jnp.where(upd, i, acci)
        if w != num_win - 1:
            accv = accv.astype(jnp.bfloat16).astype(jnp.float32)

    sem_out_ref[...] = acci.reshape(1, 1, TOK_BLK)


@jax.jit
def kernel(x, embedding_sum, cluster_usage):
    B, D, T = x.shape

    emb = embedding_sum / jnp.clip(cluster_usage, EPS, None)[:, None]
    flat = jnp.transpose(x[:, :SEM_DIM, :], (0, 2, 1)).reshape(B * T, SEM_DIM)
    x2 = jnp.sum(flat * flat, axis=1)
    e2 = jnp.sum(emb * emb, axis=1)
    ebf = emb

    toks_per_row = T // TOK_BLK
    num_tok_blocks = (B * T) // TOK_BLK
    x2_3d = x2.reshape(num_tok_blocks, 1, TOK_BLK)
    e2_2d = e2.reshape(1, SEM_K)

    sem_codes, ac_codes = pl.pallas_call(
        _vq_kernel,
        grid=(num_tok_blocks,),
        in_specs=[
            pl.BlockSpec((1, D, TOK_BLK),
                         lambda t: (t // toks_per_row, 0, t % toks_per_row)),
            pl.BlockSpec((SEM_K, SEM_DIM), lambda t: (0, 0)),
            pl.BlockSpec((1, SEM_K), lambda t: (0, 0)),
            pl.BlockSpec((1, 1, TOK_BLK), lambda t: (t, 0, 0)),
        ],
        out_specs=[
            pl.BlockSpec((1, 1, TOK_BLK),
                         lambda t: (t // toks_per_row, 0, t % toks_per_row)),
            pl.BlockSpec((1, AC_DIM, TOK_BLK),
                         lambda t: (t // toks_per_row, 0, t % toks_per_row)),
        ],
        out_shape=[
            jax.ShapeDtypeStruct((B, 1, T), jnp.int32),
            jax.ShapeDtypeStruct((B, AC_DIM, T), jnp.int32),
        ],
    )(x, ebf, e2_2d, x2_3d)

    return jnp.concatenate([sem_codes, ac_codes], axis=1)


# speedup vs baseline: 1.1278x; 1.0772x over previous
import functools

import jax
import jax.numpy as jnp
from jax.experimental import pallas as pl
from jax.experimental.pallas import tpu as pltpu

SEM_DIM = 256
AC_DIM = 36
SEM_K = 8192
AC_LEVELS = 21
EPS = 1e-05

TOK_BLK = 1024
ARGMIN_WIN = 2048


def _vq_kernel(x_ref, eb_ref, e2_ref, x2_ref,
               sem_out_ref, ac_out_ref):
    t = jnp.tanh(x_ref[0, SEM_DIM:])
    scaled = (t + 1.0) * (0.5 * (AC_LEVELS - 1))
    ac_out_ref[...] = jnp.round(scaled).astype(jnp.int32).reshape(
        1, AC_DIM, TOK_BLK)

    xb = x_ref[0, :SEM_DIM]
    a2 = (2.0 * xb).astype(jnp.bfloat16).astype(jnp.float32)
    x2 = x2_ref[0, 0]

    accv = jnp.full((TOK_BLK,), jnp.inf, jnp.float32)
    acci = jnp.zeros((TOK_BLK,), jnp.int32)
    num_win = (SEM_K + ARGMIN_WIN - 1) // ARGMIN_WIN
    for w in range(num_win):
        lo = w * ARGMIN_WIN
        wlen = min(ARGMIN_WIN, SEM_K - lo)
        ebw = eb_ref[pl.ds(lo, wlen), :]
        g2 = jax.lax.dot_general(
            ebw, a2, (((1,), (0,)), ((), ())),
            preferred_element_type=jnp.float32,
        )
        e2w = e2_ref[0, pl.ds(lo, wlen)]
        d2 = (x2[None, :] + e2w[:, None]) - g2
        dist = jnp.sqrt(jnp.maximum(0.0, d2))
        m = jnp.min(dist, axis=0)
        i = jnp.argmin(dist, axis=0) + lo
        upd = m < accv
        accv = jnp.where(upd, m, accv)
        acci = jnp.where(upd, i, acci)
        if w != num_win - 1:
            accv = accv.astype(jnp.bfloat16).astype(jnp.float32)

    sem_out_ref[...] = acci.reshape(1, 1, TOK_BLK)


@jax.jit
def kernel(x, embedding_sum, cluster_usage):
    B, D, T = x.shape

    emb = embedding_sum / jnp.clip(cluster_usage, EPS, None)[:, None]
    flat = jnp.transpose(x[:, :SEM_DIM, :], (0, 2, 1)).reshape(B * T, SEM_DIM)
    x2 = jnp.sum(flat * flat, axis=1)
    e2 = jnp.sum(emb * emb, axis=1)
    ebf = emb

    toks_per_row = T // TOK_BLK
    num_tok_blocks = (B * T) // TOK_BLK
    x2_3d = x2.reshape(num_tok_blocks, 1, TOK_BLK)
    e2_2d = e2.reshape(1, SEM_K)

    sem_codes, ac_codes = pl.pallas_call(
        _vq_kernel,
        grid=(num_tok_blocks,),
        in_specs=[
            pl.BlockSpec((1, D, TOK_BLK),
                         lambda t: (t // toks_per_row, 0, t % toks_per_row)),
            pl.BlockSpec((SEM_K, SEM_DIM), lambda t: (0, 0)),
            pl.BlockSpec((1, SEM_K), lambda t: (0, 0)),
            pl.BlockSpec((1, 1, TOK_BLK), lambda t: (t, 0, 0)),
        ],
        out_specs=[
            pl.BlockSpec((1, 1, TOK_BLK),
                         lambda t: (t // toks_per_row, 0, t % toks_per_row)),
            pl.BlockSpec((1, AC_DIM, TOK_BLK),
                         lambda t: (t // toks_per_row, 0, t % toks_per_row)),
        ],
        out_shape=[
            jax.ShapeDtypeStruct((B, 1, T), jnp.int32),
            jax.ShapeDtypeStruct((B, AC_DIM, T), jnp.int32),
        ],
    )(x, ebf, e2_2d, x2_3d)

    return jnp.concatenate([sem_codes, ac_codes], axis=1)
